# baseline (speedup 1.0000x reference)
import functools

import jax
import jax.numpy as jnp
from jax import lax
from jax.experimental import pallas as pl
from jax.experimental.pallas import tpu as pltpu
from jax.experimental.pallas import tpu_sc as plsc

VOCAB_SIZE = 30522
K_DIM = 1024
TOPK_N = 32
NOUT = 18
BATCH = 256
SEQ = 64

NC = 2
NS = 16
NW = NC * NS
K_PER_W = K_DIM // NW
ROW_W = 2 * VOCAB_SIZE
NBG = BATCH // 16


def _sc_activations(W, ids_flat):
    mesh = plsc.VectorSubcoreMesh(core_axis_name="c", subcore_axis_name="s")

    @functools.partial(
        pl.kernel,
        out_type=jax.ShapeDtypeStruct((K_DIM, BATCH), jnp.float32),
        mesh=mesh,
        compiler_params=pltpu.CompilerParams(needs_layout_passes=False),
        scratch_types=[
            pltpu.VMEM((ROW_W,), jnp.float32),
            pltpu.VMEM((SEQ * BATCH,), jnp.int32),
            pltpu.VMEM((BATCH,), jnp.float32),
        ],
    )
    def sc_act(w_hbm, ids_hbm, out_hbm, row_v, ids_v, orow_v):
        wid = lax.axis_index("s") * NC + lax.axis_index("c")
        pltpu.sync_copy(ids_hbm, ids_v)
        zidx = jnp.zeros((16,), jnp.int32)
        zval = jnp.zeros((16,), jnp.float32)

        def do_row(kl, _):
            kg = wid * K_PER_W + kl
            pltpu.sync_copy(w_hbm.at[kg], row_v)
            plsc.store_scatter(row_v, [zidx], zval)
            plsc.store_scatter(row_v, [zidx + VOCAB_SIZE], zval)

            def do_bg(bg, _):
                def do_l(l, acc):
                    idx = ids_v[pl.ds(l * BATCH + bg * 16, 16)]
                    g1 = plsc.load_gather(row_v, [idx])
                    g2 = plsc.load_gather(row_v, [idx + VOCAB_SIZE])
                    return acc + g1 + g2

                acc = lax.fori_loop(
                    0, SEQ, do_l, jnp.zeros((16,), jnp.float32))
                orow_v[pl.ds(bg * 16, 16)] = acc
                return 0

            lax.fori_loop(0, NBG, do_bg, 0)
            pltpu.sync_copy(orow_v, out_hbm.at[kg])
            return 0

        lax.fori_loop(0, K_PER_W, do_row, 0)

    return sc_act(W, ids_flat)


def _tc_topk_head(act_t, A_w, A_b2):
    def body(act_ref, aw_ref, ab_ref, out_ref, a_scr, bh_scr):
        a_scr[...] = act_ref[...]
        bh_scr[...] = jnp.zeros((K_DIM, BATCH), jnp.float32)
        iota0 = lax.broadcasted_iota(jnp.int32, (K_DIM, BATCH), 0)

        def it(i, carry):
            a = a_scr[...]
            m = jnp.max(a, axis=0, keepdims=True)
            cand = jnp.where(a == m, iota0, K_DIM)
            idx = jnp.min(cand, axis=0, keepdims=True)
            sel = iota0 == idx
            bh_scr[...] = bh_scr[...] + sel.astype(jnp.float32)
            a_scr[...] = jnp.where(sel, -jnp.inf, a)
            return carry

        lax.fori_loop(0, TOPK_N, it, 0)
        out_ref[...] = (
            lax.dot_general(
                bh_scr[...], aw_ref[...],
                (((0,), (1,)), ((), ())),
                preferred_element_type=jnp.float32,
            )
            + ab_ref[...]
        )

    return pl.pallas_call(
        body,
        out_shape=jax.ShapeDtypeStruct((BATCH, NOUT), jnp.float32),
        scratch_shapes=[
            pltpu.VMEM((K_DIM, BATCH), jnp.float32),
            pltpu.VMEM((K_DIM, BATCH), jnp.float32),
        ],
    )(act_t, A_w, A_b2)


def kernel(obs, W, A_w, A_b):
    ids = obs[:, 0, :].astype(jnp.int32)
    ids_flat = ids.T.reshape(-1)
    act_t = _sc_activations(W, ids_flat)
    logits = _tc_topk_head(act_t, A_w, A_b.reshape(1, NOUT))
    return logits


# speedup vs baseline: 3.4505x; 3.4505x over previous
import functools

import jax
import jax.numpy as jnp
from jax import lax
from jax.experimental import pallas as pl
from jax.experimental.pallas import tpu as pltpu
from jax.experimental.pallas import tpu_sc as plsc

VOCAB_SIZE = 30522
K_DIM = 1024
TOPK_N = 32
NOUT = 18
BATCH = 256
SEQ = 64
IDS_PER_B = 2 * SEQ

NC = 2
NS = 16
NW = NC * NS
B_PER_W = BATCH // NW
CHUNK = 32
NCHUNK = IDS_PER_B // CHUNK
NVREG = K_DIM // 16


def _sc_activations(W_t, idx_cat):
    mesh = plsc.VectorSubcoreMesh(core_axis_name="c", subcore_axis_name="s")

    @functools.partial(
        pl.kernel,
        out_type=jax.ShapeDtypeStruct((BATCH, K_DIM), jnp.float32),
        mesh=mesh,
        compiler_params=pltpu.CompilerParams(needs_layout_passes=False),
        scratch_types=[
            pltpu.VMEM((IDS_PER_B,), jnp.int32),
            pltpu.VMEM((2, CHUNK, K_DIM), jnp.float32),
            pltpu.VMEM((K_DIM,), jnp.float32),
            pltpu.SemaphoreType.DMA,
            pltpu.SemaphoreType.DMA,
        ],
    )
    def sc_act(wt_hbm, idx_hbm, out_hbm, idx_v, rows_v, acc_v, sem0, sem1):
        wid = lax.axis_index("s") * NC + lax.axis_index("c")
        sems = (sem0, sem1)

        def do_b(i, _):
            b = wid * B_PER_W + i
            pltpu.sync_copy(idx_hbm.at[b], idx_v)
            pltpu.async_copy(
                wt_hbm.at[idx_v.at[pl.ds(0, CHUNK)]], rows_v.at[0], sems[0]
            )
            for c in range(NCHUNK):
                buf = c % 2
                nbuf = 1 - buf
                if c + 1 < NCHUNK:
                    pltpu.async_copy(
                        wt_hbm.at[idx_v.at[pl.ds((c + 1) * CHUNK, CHUNK)]],
                        rows_v.at[nbuf],
                        sems[nbuf],
                    )
                pltpu.make_async_copy(
                    wt_hbm.at[idx_v.at[pl.ds(c * CHUNK, CHUNK)]],
                    rows_v.at[buf],
                    sems[buf],
                ).wait()

                def do_v(v, _, c=c, buf=buf):
                    o = v * 16
                    s = rows_v[buf, 0, pl.ds(o, 16)]
                    for j in range(1, CHUNK):
                        s = s + rows_v[buf, j, pl.ds(o, 16)]
                    if c == 0:
                        acc_v[pl.ds(o, 16)] = s
                    else:
                        acc_v[pl.ds(o, 16)] = acc_v[pl.ds(o, 16)] + s
                    return 0

                lax.fori_loop(0, NVREG, do_v, 0)
            pltpu.sync_copy(acc_v, out_hbm.at[b])
            return 0

        lax.fori_loop(0, B_PER_W, do_b, 0)

    return sc_act(W_t, idx_cat)


def _tc_topk_head(act, npad_col, w0_row, wv_row, A_w, A_b2):
    def body(act_ref, npad_ref, w0_ref, wv_ref, aw_ref, ab_ref, out_ref,
             a_scr, bh_scr):
        corr = npad_ref[...] * (w0_ref[...] + wv_ref[...])
        a_scr[...] = act_ref[...] - corr
        bh_scr[...] = jnp.zeros((BATCH, K_DIM), jnp.float32)
        iota1 = lax.broadcasted_iota(jnp.int32, (BATCH, K_DIM), 1)

        def it(i, carry):
            a = a_scr[...]
            m = jnp.max(a, axis=1, keepdims=True)
            cand = jnp.where(a == m, iota1, K_DIM)
            idx = jnp.min(cand, axis=1, keepdims=True)
            sel = iota1 == idx
            bh_scr[...] = bh_scr[...] + sel.astype(jnp.float32)
            a_scr[...] = jnp.where(sel, -jnp.inf, a)
            return carry

        lax.fori_loop(0, TOPK_N, it, 0)
        out_ref[...] = (
            lax.dot_general(
                bh_scr[...], aw_ref[...],
                (((1,), (1,)), ((), ())),
                preferred_element_type=jnp.float32,
            )
            + ab_ref[...]
        )

    return pl.pallas_call(
        body,
        out_shape=jax.ShapeDtypeStruct((BATCH, NOUT), jnp.float32),
        scratch_shapes=[
            pltpu.VMEM((BATCH, K_DIM), jnp.float32),
            pltpu.VMEM((BATCH, K_DIM), jnp.float32),
        ],
    )(act, npad_col, w0_row, wv_row, A_w, A_b2)


def kernel(obs, W, A_w, A_b):
    ids = obs[:, 0, :].astype(jnp.int32)
    idx_cat = jnp.concatenate([ids, ids + VOCAB_SIZE], axis=1)
    W_t = W.T
    act = _sc_activations(W_t, idx_cat)
    npad_col = jnp.sum((ids == 0).astype(jnp.float32), axis=1, keepdims=True)
    w0_row = W[:, 0].reshape(1, K_DIM)
    wv_row = W[:, VOCAB_SIZE].reshape(1, K_DIM)
    logits = _tc_topk_head(
        act, npad_col, w0_row, wv_row, A_w, A_b.reshape(1, NOUT))
    return logits


# speedup vs baseline: 4.3939x; 1.2734x over previous
import functools

import jax
import jax.numpy as jnp
from jax import lax
from jax.experimental import pallas as pl
from jax.experimental.pallas import tpu as pltpu
from jax.experimental.pallas import tpu_sc as plsc

VOCAB_SIZE = 30522
K_DIM = 1024
TOPK_N = 32
NOUT = 18
BATCH = 256
SEQ = 64
IDS_PER_B = 2 * SEQ

NC = 2
NS = 16
NW = NC * NS
B_PER_W = BATCH // NW
CHUNK = 32
NCHUNK = IDS_PER_B // CHUNK
NVREG = K_DIM // 16


def _sc_activations(W_t, idx_cat):
    mesh = plsc.VectorSubcoreMesh(core_axis_name="c", subcore_axis_name="s")

    @functools.partial(
        pl.kernel,
        out_type=jax.ShapeDtypeStruct((BATCH, K_DIM), jnp.float32),
        mesh=mesh,
        compiler_params=pltpu.CompilerParams(needs_layout_passes=False),
        scratch_types=[
            pltpu.VMEM((B_PER_W, IDS_PER_B), jnp.int32),
            pltpu.VMEM((2, CHUNK, K_DIM), jnp.float32),
            pltpu.VMEM((B_PER_W, K_DIM), jnp.float32),
            pltpu.SemaphoreType.DMA,
            pltpu.SemaphoreType.DMA,
            pltpu.SemaphoreType.DMA,
        ],
    )
    def sc_act(wt_hbm, idx_hbm, out_hbm, idx_v, rows_v, acc_v, sem0, sem1,
               semo):
        wid = lax.axis_index("s") * NC + lax.axis_index("c")
        b0 = wid * B_PER_W
        sems = (sem0, sem1)
        pltpu.sync_copy(idx_hbm.at[pl.ds(b0, B_PER_W)], idx_v)

        NSTEP = B_PER_W * NCHUNK

        def gather(s, buf):
            i, c = divmod(s, NCHUNK)
            pltpu.async_copy(
                wt_hbm.at[idx_v.at[i, pl.ds(c * CHUNK, CHUNK)]],
                rows_v.at[buf],
                sems[buf],
            )

        gather(0, 0)
        for s in range(NSTEP):
            i, c = divmod(s, NCHUNK)
            buf = s % 2
            if s + 1 < NSTEP:
                gather(s + 1, 1 - buf)
            pltpu.make_async_copy(
                wt_hbm.at[idx_v.at[i, pl.ds(c * CHUNK, CHUNK)]],
                rows_v.at[buf],
                sems[buf],
            ).wait()

            def do_v(v, _, i=i, c=c, buf=buf):
                o = v * 16
                s = rows_v[buf, 0, pl.ds(o, 16)]
                for j in range(1, CHUNK):
                    s = s + rows_v[buf, j, pl.ds(o, 16)]
                if c == 0:
                    acc_v[i, pl.ds(o, 16)] = s
                else:
                    acc_v[i, pl.ds(o, 16)] = acc_v[i, pl.ds(o, 16)] + s
                return 0

            lax.fori_loop(0, NVREG, do_v, 0)
            if c == NCHUNK - 1:
                pltpu.async_copy(acc_v.at[i], out_hbm.at[b0 + i], semo)
        for i in range(B_PER_W):
            pltpu.make_async_copy(
                acc_v.at[i], out_hbm.at[b0 + i], semo).wait()

    return sc_act(W_t, idx_cat)


def _tc_topk_head(act, npad_col, w0_row, wv_row, A_w, A_b2):
    def body(act_ref, npad_ref, w0_ref, wv_ref, aw_ref, ab_ref, out_ref,
             a_scr, bh_scr):
        corr = npad_ref[...] * (w0_ref[...] + wv_ref[...])
        a_scr[...] = act_ref[...] - corr
        bh_scr[...] = jnp.zeros((BATCH, K_DIM), jnp.float32)
        iota1 = lax.broadcasted_iota(jnp.int32, (BATCH, K_DIM), 1)

        def it(i, carry):
            a = a_scr[...]
            m = jnp.max(a, axis=1, keepdims=True)
            cand = jnp.where(a == m, iota1, K_DIM)
            idx = jnp.min(cand, axis=1, keepdims=True)
            sel = iota1 == idx
            bh_scr[...] = bh_scr[...] + sel.astype(jnp.float32)
            a_scr[...] = jnp.where(sel, -jnp.inf, a)
            return carry

        lax.fori_loop(0, TOPK_N, it, 0)
        out_ref[...] = (
            lax.dot_general(
                bh_scr[...], aw_ref[...],
                (((1,), (1,)), ((), ())),
                preferred_element_type=jnp.float32,
            )
            + ab_ref[...]
        )

    return pl.pallas_call(
        body,
        out_shape=jax.ShapeDtypeStruct((BATCH, NOUT), jnp.float32),
        scratch_shapes=[
            pltpu.VMEM((BATCH, K_DIM), jnp.float32),
            pltpu.VMEM((BATCH, K_DIM), jnp.float32),
        ],
    )(act, npad_col, w0_row, wv_row, A_w, A_b2)


def kernel(obs, W, A_w, A_b):
    ids = obs[:, 0, :].astype(jnp.int32)
    idx_cat = jnp.concatenate([ids, ids + VOCAB_SIZE], axis=1)
    W_t = W.T
    act = _sc_activations(W_t, idx_cat)
    npad_col = jnp.sum((ids == 0).astype(jnp.float32), axis=1, keepdims=True)
    w0_row = W[:, 0].reshape(1, K_DIM)
    wv_row = W[:, VOCAB_SIZE].reshape(1, K_DIM)
    logits = _tc_topk_head(
        act, npad_col, w0_row, wv_row, A_w, A_b.reshape(1, NOUT))
    return logits


# speedup vs baseline: 4.5565x; 1.0370x over previous
import functools

import jax
import jax.numpy as jnp
from jax import lax
from jax.experimental import pallas as pl
from jax.experimental.pallas import tpu as pltpu
from jax.experimental.pallas import tpu_sc as plsc

VOCAB_SIZE = 30522
K_DIM = 1024
TOPK_N = 32
NOUT = 18
BATCH = 256
SEQ = 64
IDS_PER_B = 2 * SEQ

NC = 2
NS = 16
NW = NC * NS
B_PER_W = BATCH // NW
CHUNK = 32
NCHUNK = IDS_PER_B // CHUNK
NVREG = K_DIM // 16


def _sc_activations(W_t, idx_cat):
    mesh = plsc.VectorSubcoreMesh(core_axis_name="c", subcore_axis_name="s")

    @functools.partial(
        pl.kernel,
        out_type=jax.ShapeDtypeStruct((BATCH, K_DIM), jnp.float32),
        mesh=mesh,
        compiler_params=pltpu.CompilerParams(needs_layout_passes=False),
        scratch_types=[
            pltpu.VMEM((B_PER_W, IDS_PER_B), jnp.int32),
            pltpu.VMEM((2, CHUNK, K_DIM), jnp.float32),
            pltpu.VMEM((B_PER_W, K_DIM), jnp.float32),
            pltpu.SemaphoreType.DMA,
            pltpu.SemaphoreType.DMA,
            pltpu.SemaphoreType.DMA,
        ],
    )
    def sc_act(wt_hbm, idx_hbm, out_hbm, idx_v, rows_v, acc_v, sem0, sem1,
               semo):
        wid = lax.axis_index("s") * NC + lax.axis_index("c")
        b0 = wid * B_PER_W
        sems = (sem0, sem1)
        pltpu.sync_copy(idx_hbm.at[pl.ds(b0, B_PER_W)], idx_v)

        NSTEP = B_PER_W * NCHUNK

        def gather(s, buf):
            i, c = divmod(s, NCHUNK)
            pltpu.async_copy(
                wt_hbm.at[idx_v.at[i, pl.ds(c * CHUNK, CHUNK)]],
                rows_v.at[buf],
                sems[buf],
            )

        gather(0, 0)
        for s in range(NSTEP):
            i, c = divmod(s, NCHUNK)
            buf = s % 2
            if s + 1 < NSTEP:
                gather(s + 1, 1 - buf)
            pltpu.make_async_copy(
                wt_hbm.at[idx_v.at[i, pl.ds(c * CHUNK, CHUNK)]],
                rows_v.at[buf],
                sems[buf],
            ).wait()

            def do_v(v, _, i=i, c=c, buf=buf):
                o = v * 16
                s = rows_v[buf, 0, pl.ds(o, 16)]
                for j in range(1, CHUNK):
                    s = s + rows_v[buf, j, pl.ds(o, 16)]
                if c == 0:
                    acc_v[i, pl.ds(o, 16)] = s
                else:
                    acc_v[i, pl.ds(o, 16)] = acc_v[i, pl.ds(o, 16)] + s
                return 0

            lax.fori_loop(0, NVREG, do_v, 0)
            if c == NCHUNK - 1:
                pltpu.async_copy(acc_v.at[i], out_hbm.at[b0 + i], semo)
        for i in range(B_PER_W):
            pltpu.make_async_copy(
                acc_v.at[i], out_hbm.at[b0 + i], semo).wait()

    return sc_act(W_t, idx_cat)


def _tc_topk_head(act, npad_col, w0_row, wv_row, A_w, A_b2):
    def body(act_ref, npad_ref, w0_ref, wv_ref, aw_ref, ab_ref, out_ref,
             a_scr):
        corr = npad_ref[...] * (w0_ref[...] + wv_ref[...])
        a_scr[...] = act_ref[...] - corr
        iota1 = lax.broadcasted_iota(jnp.int32, (BATCH, K_DIM), 1)

        for _ in range(TOPK_N):
            a = a_scr[...]
            m = jnp.max(a, axis=1, keepdims=True)
            cand = jnp.where(a == m, iota1, K_DIM)
            idx = jnp.min(cand, axis=1, keepdims=True)
            a_scr[...] = jnp.where(iota1 == idx, -jnp.inf, a)

        bh = (a_scr[...] == -jnp.inf).astype(jnp.float32)
        out_ref[...] = (
            lax.dot_general(
                bh, aw_ref[...],
                (((1,), (1,)), ((), ())),
                preferred_element_type=jnp.float32,
            )
            + ab_ref[...]
        )

    return pl.pallas_call(
        body,
        out_shape=jax.ShapeDtypeStruct((BATCH, NOUT), jnp.float32),
        scratch_shapes=[
            pltpu.VMEM((BATCH, K_DIM), jnp.float32),
        ],
    )(act, npad_col, w0_row, wv_row, A_w, A_b2)


def kernel(obs, W, A_w, A_b):
    ids = obs[:, 0, :].astype(jnp.int32)
    idx_cat = jnp.concatenate([ids, ids + VOCAB_SIZE], axis=1)
    W_t = W.T
    act = _sc_activations(W_t, idx_cat)
    npad_col = jnp.sum((ids == 0).astype(jnp.float32), axis=1, keepdims=True)
    w0_row = W[:, 0].reshape(1, K_DIM)
    wv_row = W[:, VOCAB_SIZE].reshape(1, K_DIM)
    logits = _tc_topk_head(
        act, npad_col, w0_row, wv_row, A_w, A_b.reshape(1, NOUT))
    return logits
